# baseline (speedup 1.0000x reference)
import functools

import jax
import jax.numpy as jnp
from jax import lax
from jax.experimental import pallas as pl
from jax.experimental.pallas import tpu as pltpu
from jax.experimental.pallas import tpu_sc as plsc


@functools.lru_cache(maxsize=None)
def _make_copy_kernel(n_total: int):
    info = plsc.get_sparse_core_info()
    nc, ns = info.num_cores, info.num_subcores
    nw = nc * ns
    assert n_total % nw == 0
    chunk = n_total // nw
    mesh = plsc.VectorSubcoreMesh(core_axis_name="c", subcore_axis_name="s")

    @functools.partial(
        pl.kernel,
        mesh=mesh,
        out_type=jax.ShapeDtypeStruct((n_total,), jnp.float32),
    )
    def copy_k(src_hbm, out_hbm):
        wid = lax.axis_index("s") * nc + lax.axis_index("c")
        base = wid * chunk
        pltpu.sync_copy(src_hbm.at[pl.ds(base, chunk)],
                        out_hbm.at[pl.ds(base, chunk)])

    return copy_k


def kernel(inputs, pos_embedding):
    assert inputs.ndim == 3
    length = inputs.shape[1]
    _, max_len, d = pos_embedding.shape
    assert length == max_len
    flat = pos_embedding.reshape(max_len * d)
    out = _make_copy_kernel(max_len * d)(flat)
    return out.reshape(1, length, d)


# speedup vs baseline: 6.4748x; 6.4748x over previous
import functools

import jax
import jax.numpy as jnp
from jax import lax
from jax.experimental import pallas as pl
from jax.experimental.pallas import tpu as pltpu
from jax.experimental.pallas import tpu_sc as plsc


@functools.lru_cache(maxsize=None)
def _make_copy_kernel(n_total: int):
    info = plsc.get_sparse_core_info()
    nc, ns = info.num_cores, info.num_subcores
    nw = nc * ns
    assert n_total % nw == 0
    chunk = n_total // nw
    mesh = plsc.VectorSubcoreMesh(core_axis_name="c", subcore_axis_name="s")

    @functools.partial(
        pl.kernel,
        mesh=mesh,
        out_type=jax.ShapeDtypeStruct((n_total,), jnp.float32),
        scratch_types=[pltpu.VMEM((chunk,), jnp.float32)],
    )
    def copy_k(src_hbm, out_hbm, buf):
        wid = lax.axis_index("s") * nc + lax.axis_index("c")
        base = wid * chunk
        pltpu.sync_copy(src_hbm.at[pl.ds(base, chunk)], buf)
        pltpu.sync_copy(buf, out_hbm.at[pl.ds(base, chunk)])

    return copy_k


def kernel(inputs, pos_embedding):
    assert inputs.ndim == 3
    length = inputs.shape[1]
    _, max_len, d = pos_embedding.shape
    assert length == max_len
    flat = pos_embedding.reshape(max_len * d)
    out = _make_copy_kernel(max_len * d)(flat)
    return out.reshape(1, length, d)


# speedup vs baseline: 11.7666x; 1.8173x over previous
import functools

import jax
import jax.numpy as jnp
from jax import lax
from jax.experimental import pallas as pl
from jax.experimental.pallas import tpu as pltpu
from jax.experimental.pallas import tpu_sc as plsc


@functools.lru_cache(maxsize=None)
def _make_copy_kernel(rows: int, d: int):
    info = plsc.get_sparse_core_info()
    nc, ns = info.num_cores, info.num_subcores
    nw = nc * ns
    assert rows % nw == 0
    chunk = rows // nw
    mesh = plsc.VectorSubcoreMesh(core_axis_name="c", subcore_axis_name="s")

    @functools.partial(
        pl.kernel,
        mesh=mesh,
        out_type=jax.ShapeDtypeStruct((rows, d), jnp.float32),
        scratch_types=[pltpu.VMEM((chunk, d), jnp.float32)],
    )
    def copy_k(src_hbm, out_hbm, buf):
        wid = lax.axis_index("s") * nc + lax.axis_index("c")
        base = wid * chunk
        pltpu.sync_copy(src_hbm.at[pl.ds(base, chunk), :], buf)
        pltpu.sync_copy(buf, out_hbm.at[pl.ds(base, chunk), :])

    return copy_k


def kernel(inputs, pos_embedding):
    assert inputs.ndim == 3
    length = inputs.shape[1]
    _, max_len, d = pos_embedding.shape
    assert length == max_len
    out = _make_copy_kernel(max_len, d)(pos_embedding.reshape(max_len, d))
    return out.reshape(1, length, d)
